# Initial kernel scaffold; baseline (speedup 1.0000x reference)
import jax
import jax.numpy as jnp
from jax.experimental import pallas as pl


def kernel(x, pre_w, log_weight):
    raise NotImplementedError("write your pallas kernel here")


# speedup vs baseline: 10.6984x; 10.6984x over previous
import jax
import jax.numpy as jnp
from jax.experimental import pallas as pl

_K = 64
_BLK = 8


def _select_agg_body(lw_ref, pw_ref, v_ref):
    pw = pw_ref[...]
    rt, c = pw.shape
    bits = jax.lax.bitcast_convert_type(pw, jnp.uint32)
    sign = bits >> jnp.uint32(31)
    flip = jnp.where(sign > 0, jnp.uint32(0xFFFFFFFF), jnp.uint32(0x80000000))
    ukey = bits ^ flip
    kf = jnp.float32(_K)

    def sstep(i, prefix):
        bit = jnp.uint32(1) << (jnp.uint32(31) - i.astype(jnp.uint32))
        cand = prefix | bit
        cnt = jnp.sum((ukey >= cand).astype(jnp.float32), axis=1,
                      keepdims=True)
        return jnp.where(cnt >= kf, cand, prefix)

    thr = jax.lax.fori_loop(0, 32, sstep, jnp.zeros((rt, 1), jnp.uint32))

    gt = ukey > thr
    n_gt = jnp.sum(gt.astype(jnp.float32), axis=1, keepdims=True)
    need = kf - n_gt
    eq = ukey == thr
    col = jax.lax.broadcasted_iota(jnp.int32, (rt, c), 1)

    def tstep(i, prefix):
        cand = prefix | (jnp.int32(1) << (jnp.int32(11) - i))
        g = jnp.sum((eq & (col < cand)).astype(jnp.float32), axis=1,
                    keepdims=True)
        return jnp.where(g <= need, cand, prefix)

    cstar = jax.lax.fori_loop(0, 12, tstep, jnp.zeros((rt, 1), jnp.int32))
    mask = gt | (eq & (col < cstar))

    wp = jnp.where(mask, jnp.exp(pw), 0.0)

    coef = jnp.exp(lw_ref[0, 0, :])
    o_ix = jax.lax.broadcasted_iota(jnp.int32, (rt // _BLK, rt), 0)
    b_ix = jax.lax.broadcasted_iota(jnp.int32, (rt // _BLK, rt), 1)
    s = jnp.where(o_ix == (b_ix // _BLK), coef[None, :], 0.0)
    v = jax.lax.dot_general(s, wp, (((1,), (0,)), ((), ())),
                            preferred_element_type=jnp.float32)
    v_ref[...] = v.astype(jnp.bfloat16)


def _matmul_body(x_ref, v_ref, o_ref):
    xb = x_ref[...].astype(jnp.bfloat16)
    vb = v_ref[...]
    o_ref[...] = jax.lax.dot_general(
        xb, vb, (((1,), (1,)), ((), ())),
        preferred_element_type=jnp.float32)


def kernel(x, pre_w, log_weight):
    n_tokens, in_features = x.shape
    n_branches = pre_w.shape[0]
    out_features, blk = log_weight.shape
    assert blk == _BLK

    rt = 256
    n_row_tiles = n_branches // rt
    lw3 = log_weight.reshape(n_row_tiles, 1, rt)

    v = pl.pallas_call(
        _select_agg_body,
        grid=(n_row_tiles,),
        in_specs=[
            pl.BlockSpec((1, 1, rt), lambda i: (i, 0, 0)),
            pl.BlockSpec((rt, in_features), lambda i: (i, 0)),
        ],
        out_specs=pl.BlockSpec((rt // _BLK, in_features), lambda i: (i, 0)),
        out_shape=jax.ShapeDtypeStruct((out_features, in_features),
                                       jnp.bfloat16),
    )(lw3, pre_w)

    tt = 1024
    out = pl.pallas_call(
        _matmul_body,
        grid=(n_tokens // tt,),
        in_specs=[
            pl.BlockSpec((tt, in_features), lambda i: (i, 0)),
            pl.BlockSpec((out_features, in_features), lambda i: (0, 0)),
        ],
        out_specs=pl.BlockSpec((tt, out_features), lambda i: (i, 0)),
        out_shape=jax.ShapeDtypeStruct((n_tokens, out_features), jnp.float32),
    )(x, v)
    return out
